# baseline (speedup 1.0000x reference)
import functools

import jax
import jax.numpy as jnp
from jax import lax
from jax.experimental import pallas as pl
from jax.experimental.pallas import tpu as pltpu
from jax.experimental.pallas import tpu_sc as plsc

H = 16
TP = 16
T1 = 1024
T2 = 2048
B_ALL = T1 + T2

_NC = 2
_NS = 16
_NW = _NC * _NS
_B_PER_W = B_ALL // _NW


def _make_sc_gather():
    mesh = plsc.VectorSubcoreMesh(core_axis_name="c", subcore_axis_name="s")

    @functools.partial(
        pl.kernel,
        mesh=mesh,
        out_type=jax.ShapeDtypeStruct((B_ALL, 128), jnp.uint32),
        scratch_types=[
            pltpu.VMEM((_B_PER_W,), jnp.int32),
            pltpu.VMEM((_B_PER_W, 128), jnp.uint32),
            pltpu.SemaphoreType.DMA,
        ],
    )
    def sc_gather(table_hbm, sup_hbm, out_hbm, idx_v, rows_v, sem):
        wid = lax.axis_index("s") * _NC + lax.axis_index("c")
        base = wid * _B_PER_W
        pltpu.sync_copy(sup_hbm.at[pl.ds(base, _B_PER_W)], idx_v)
        pltpu.async_copy(table_hbm.at[idx_v], rows_v, sem).wait()
        pltpu.sync_copy(rows_v, out_hbm.at[pl.ds(base, _B_PER_W)])

    return sc_gather


_BI = 256
_BJ = 512


def _dist_body(norm_ref, sta_ref, post_ref, out_ref):
    sa = sta_ref[...]
    aabs = jnp.abs(sa)
    asig = (sa < 0).astype(jnp.int32)
    pt = post_ref[...]
    pabs = jnp.abs(pt)
    psig = (pt < 0).astype(jnp.int32)
    acc = jnp.zeros((_BI, _BJ), jnp.int32)
    for k in range(TP):
        x = aabs[:, k : k + 1] ^ pabs[k : k + 1, :]
        v = (x + 1).astype(jnp.float32)
        e = lax.bitcast_convert_type(v, jnp.int32) >> 23
        q = 142 - e
        sx = asig[:, k : k + 1] ^ psig[k : k + 1, :]
        acc = acc + ((q ^ (-sx)) + sx)
    out_ref[...] = acc.astype(jnp.float32) * (norm_ref[...] * (1.0 / 256.0))


def _tc_distance(norm, sta, pos_t, interpret=False):
    grid = (T1 // _BI, T2 // _BJ)
    return pl.pallas_call(
        _dist_body,
        grid=grid,
        in_specs=[
            pl.BlockSpec((_BI, _BJ), lambda i, j: (i, j)),
            pl.BlockSpec((_BI, TP), lambda i, j: (i, jnp.int32(0))),
            pl.BlockSpec((TP, _BJ), lambda i, j: (jnp.int32(0), j)),
        ],
        out_specs=pl.BlockSpec((_BI, _BJ), lambda i, j: (i, j)),
        out_shape=jax.ShapeDtypeStruct((T1, T2), jnp.float32),
        interpret=interpret,
    )(norm, sta, pos_t)


def kernel(norm, sta_idx, pos_idx, locations):
    loc_w = jax.lax.bitcast_convert_type(locations, jnp.uint32)
    loc_w = loc_w.reshape(locations.shape[0] // 4, 128)
    idx_all = jnp.concatenate([sta_idx, pos_idx]).astype(jnp.int32)
    sup = idx_all >> 2
    off = (idx_all & 3) * 32
    rows128 = _make_sc_gather()(loc_w, sup)
    cols = off[:, None] + 2 * jnp.arange(TP, dtype=jnp.int32)[None, :]
    rows32 = jnp.take_along_axis(rows128, cols, axis=1).astype(jnp.int32)
    sta = rows32[:T1]
    pos_t = rows32[T1:].T
    return _tc_distance(norm, sta, pos_t)


# speedup vs baseline: 35.4195x; 35.4195x over previous
import functools

import jax
import jax.numpy as jnp
from jax import lax
from jax.experimental import pallas as pl
from jax.experimental.pallas import tpu as pltpu
from jax.experimental.pallas import tpu_sc as plsc

H = 16
TP = 16
T1 = 1024
T2 = 2048
B_ALL = T1 + T2

_NC = 2
_NS = 16
_B_PER_TEC = 128
_N_ACTIVE = B_ALL // _B_PER_TEC


def _make_sc_gather():
    mesh = plsc.VectorSubcoreMesh(core_axis_name="c", subcore_axis_name="s")

    @functools.partial(
        pl.kernel,
        mesh=mesh,
        out_type=jax.ShapeDtypeStruct((TP, B_ALL), jnp.int32),
        scratch_types=[
            pltpu.VMEM((_B_PER_TEC,), jnp.int32),
            pltpu.VMEM((16, TP, 128), jnp.int32),
            pltpu.VMEM((TP, _B_PER_TEC), jnp.int32),
            pltpu.SemaphoreType.DMA,
        ],
        compiler_params=pltpu.CompilerParams(needs_layout_passes=False),
    )
    def sc_gather(table_hbm, idx_hbm, out_hbm, idx_v, bufs, outT_v, sem):
        wid = lax.axis_index("s") * _NC + lax.axis_index("c")

        @pl.when(wid < _N_ACTIVE)
        def _():
            base = wid * _B_PER_TEC
            pltpu.sync_copy(idx_hbm.at[pl.ds(base, _B_PER_TEC)], idx_v)
            lane = jnp.arange(16, dtype=jnp.int32)
            for c in range(_B_PER_TEC // 16):
                chunk = idx_v[pl.ds(c * 16, 16)]
                tcol = (chunk >> 7) * 128
                loff = chunk & 127
                for j in range(16):
                    bj = jnp.sum(jnp.where(lane == j, tcol, 0),
                                 dtype=jnp.int32)
                    bj = pl.multiple_of(bj, 128)
                    pltpu.make_async_copy(
                        table_hbm.at[pl.ds(jnp.int32(0), TP), pl.ds(bj, 128)],
                        bufs.at[jnp.int32(j)], sem,
                    ).start()
                for j in range(16):
                    pltpu.make_async_copy(
                        table_hbm.at[pl.ds(jnp.int32(0), TP),
                                     pl.ds(jnp.int32(0), 128)],
                        bufs.at[jnp.int32(j)], sem,
                    ).wait()
                for k in range(TP):
                    g = plsc.load_gather(
                        bufs, [lane, jnp.full_like(lane, k), loff]
                    )
                    outT_v[k, pl.ds(c * 16, 16)] = g
            pltpu.sync_copy(outT_v, out_hbm.at[:, pl.ds(base, _B_PER_TEC)])

    return sc_gather


_BI = 256
_BJ = 512


def _dist_body(norm_ref, sta_ref, post_ref, out_ref):
    sa = sta_ref[...]
    aabs = jnp.abs(sa)
    asig = sa >> 31
    pt = post_ref[...]
    pabs = jnp.abs(pt)
    psig = pt >> 31
    acc = jnp.zeros((_BI, _BJ), jnp.int32)
    for k in range(TP):
        x = aabs[:, k : k + 1] ^ pabs[k : k + 1, :]
        v = (x + 1).astype(jnp.float32)
        e = lax.bitcast_convert_type(v, jnp.int32) >> 23
        q = 142 - e
        m = asig[:, k : k + 1] ^ psig[k : k + 1, :]
        acc = acc + ((q ^ m) - m)
    out_ref[...] = acc.astype(jnp.float32) * (norm_ref[...] * (1.0 / 256.0))


def _tc_distance(norm, sta, rows_t, interpret=False):
    grid = (T1 // _BI, T2 // _BJ)
    return pl.pallas_call(
        _dist_body,
        grid=grid,
        in_specs=[
            pl.BlockSpec((_BI, _BJ), lambda i, j: (i, j)),
            pl.BlockSpec((_BI, TP), lambda i, j: (i, jnp.int32(0))),
            pl.BlockSpec((TP, _BJ), lambda i, j: (jnp.int32(0), j + T1 // _BJ)),
        ],
        out_specs=pl.BlockSpec((_BI, _BJ), lambda i, j: (i, j)),
        out_shape=jax.ShapeDtypeStruct((T1, T2), jnp.float32),
        interpret=interpret,
    )(norm, sta, rows_t)


def kernel(norm, sta_idx, pos_idx, locations):
    loc32_t = locations.astype(jnp.int32).T
    idx_all = jnp.concatenate([sta_idx, pos_idx]).astype(jnp.int32)
    rows_t = _make_sc_gather()(loc32_t, idx_all)
    sta = rows_t[:, :T1].T
    return _tc_distance(norm, sta, rows_t)


# speedup vs baseline: 37.9105x; 1.0703x over previous
import functools

import jax
import jax.numpy as jnp
from jax import lax
from jax.experimental import pallas as pl
from jax.experimental.pallas import tpu as pltpu
from jax.experimental.pallas import tpu_sc as plsc

H = 16
TP = 16
T1 = 1024
T2 = 2048
B_ALL = T1 + T2

_NC = 2
_NS = 16
_B_PER_TEC = 128
_N_ACTIVE = B_ALL // _B_PER_TEC


def _make_sc_gather():
    mesh = plsc.VectorSubcoreMesh(core_axis_name="c", subcore_axis_name="s")

    @functools.partial(
        pl.kernel,
        mesh=mesh,
        out_type=jax.ShapeDtypeStruct((TP, B_ALL), jnp.int32),
        scratch_types=[
            pltpu.VMEM((_B_PER_TEC,), jnp.int32),
            pltpu.VMEM((16, TP, 128), jnp.int32),
            pltpu.VMEM((TP, _B_PER_TEC), jnp.int32),
            pltpu.SemaphoreType.DMA,
        ],
        compiler_params=pltpu.CompilerParams(needs_layout_passes=False),
    )
    def sc_gather(table_hbm, idx_hbm, out_hbm, idx_v, bufs, outT_v, sem):
        wid = lax.axis_index("s") * _NC + lax.axis_index("c")

        @pl.when(wid < _N_ACTIVE)
        def _():
            base = wid * _B_PER_TEC
            pltpu.sync_copy(idx_hbm.at[pl.ds(base, _B_PER_TEC)], idx_v)
            lane = jnp.arange(16, dtype=jnp.int32)
            for c in range(_B_PER_TEC // 16):
                chunk = idx_v[pl.ds(c * 16, 16)]
                tcol = (chunk >> 7) * 128
                loff = chunk & 127
                for j in range(16):
                    bj = jnp.sum(jnp.where(lane == j, tcol, 0),
                                 dtype=jnp.int32)
                    bj = pl.multiple_of(bj, 128)
                    pltpu.make_async_copy(
                        table_hbm.at[pl.ds(jnp.int32(0), TP), pl.ds(bj, 128)],
                        bufs.at[jnp.int32(j)], sem,
                    ).start()
                for j in range(16):
                    pltpu.make_async_copy(
                        table_hbm.at[pl.ds(jnp.int32(0), TP),
                                     pl.ds(jnp.int32(0), 128)],
                        bufs.at[jnp.int32(j)], sem,
                    ).wait()
                for k in range(TP):
                    g = plsc.load_gather(
                        bufs, [lane, jnp.full_like(lane, k), loff]
                    )
                    outT_v[k, pl.ds(c * 16, 16)] = g
            pltpu.sync_copy(outT_v, out_hbm.at[:, pl.ds(base, _B_PER_TEC)])

    return sc_gather


_BI = 256
_BJ = 512


def _dist_body(norm_ref, sta_ref, post_ref, out_ref):
    sa = sta_ref[...]
    aabs = jnp.abs(sa)
    asig = sa >> 31
    pt = post_ref[...]
    pabs = jnp.abs(pt)
    psig = pt >> 31
    acc = jnp.zeros((_BI, _BJ), jnp.int32)
    for k in range(TP):
        x = aabs[:, k : k + 1] ^ pabs[k : k + 1, :]
        v = (x + 1).astype(jnp.float32)
        e = lax.bitcast_convert_type(v, jnp.int32) >> 23
        q = 142 - e
        m = asig[:, k : k + 1] ^ psig[k : k + 1, :]
        acc = acc + ((q ^ m) - m)
    out_ref[...] = acc.astype(jnp.float32) * (norm_ref[...] * (1.0 / 256.0))


def _tc_distance(norm, sta, rows_t, interpret=False):
    grid = (T1 // _BI, T2 // _BJ)
    return pl.pallas_call(
        _dist_body,
        grid=grid,
        in_specs=[
            pl.BlockSpec((_BI, _BJ), lambda i, j: (i, j)),
            pl.BlockSpec((_BI, TP), lambda i, j: (i, jnp.int32(0))),
            pl.BlockSpec((TP, _BJ), lambda i, j: (jnp.int32(0), j + T1 // _BJ)),
        ],
        out_specs=pl.BlockSpec((_BI, _BJ), lambda i, j: (i, j)),
        out_shape=jax.ShapeDtypeStruct((T1, T2), jnp.float32),
        interpret=interpret,
    )(norm, sta, rows_t)


def kernel(norm, sta_idx, pos_idx, locations):
    loc_u_t = locations.astype(jnp.uint32).T
    idx_all = jnp.concatenate([sta_idx, pos_idx]).astype(jnp.int32)
    rows_t = _make_sc_gather()(loc_u_t, idx_all)
    sta = rows_t[:, :T1].T
    return _tc_distance(norm, sta, rows_t)


# speedup vs baseline: 38.2101x; 1.0079x over previous
import functools

import jax
import jax.numpy as jnp
from jax import lax
from jax.experimental import pallas as pl
from jax.experimental.pallas import tpu as pltpu
from jax.experimental.pallas import tpu_sc as plsc

H = 16
TP = 16
T1 = 1024
T2 = 2048
B_ALL = T1 + T2

_NC = 2
_NS = 16
_B_PER_TEC = 128
_N_ACTIVE = B_ALL // _B_PER_TEC


def _make_sc_gather():
    mesh = plsc.VectorSubcoreMesh(core_axis_name="c", subcore_axis_name="s")

    @functools.partial(
        pl.kernel,
        mesh=mesh,
        out_type=jax.ShapeDtypeStruct((TP, B_ALL), jnp.int32),
        scratch_types=[
            pltpu.VMEM((_B_PER_TEC,), jnp.int32),
            pltpu.VMEM((16, TP, 128), jnp.int32),
            pltpu.VMEM((TP, _B_PER_TEC), jnp.int32),
            pltpu.SemaphoreType.DMA,
        ],
        compiler_params=pltpu.CompilerParams(needs_layout_passes=False),
    )
    def sc_gather(table_hbm, idx_hbm, out_hbm, idx_v, bufs, outT_v, sem):
        wid = lax.axis_index("s") * _NC + lax.axis_index("c")

        @pl.when(wid < _N_ACTIVE)
        def _():
            base = wid * _B_PER_TEC
            pltpu.sync_copy(idx_hbm.at[pl.ds(base, _B_PER_TEC)], idx_v)
            lane = jnp.arange(16, dtype=jnp.int32)
            for c in range(_B_PER_TEC // 16):
                chunk = idx_v[pl.ds(c * 16, 16)]
                tcol = (chunk >> 7) * 128
                loff = chunk & 127
                for j in range(16):
                    bj = jnp.sum(jnp.where(lane == j, tcol, 0),
                                 dtype=jnp.int32)
                    bj = pl.multiple_of(bj, 128)
                    pltpu.make_async_copy(
                        table_hbm.at[pl.ds(jnp.int32(0), TP), pl.ds(bj, 128)],
                        bufs.at[jnp.int32(j)], sem,
                    ).start()
                for j in range(16):
                    pltpu.make_async_copy(
                        table_hbm.at[pl.ds(jnp.int32(0), TP),
                                     pl.ds(jnp.int32(0), 128)],
                        bufs.at[jnp.int32(j)], sem,
                    ).wait()
                for k in range(TP):
                    g = plsc.load_gather(
                        bufs, [lane, jnp.full_like(lane, k), loff]
                    )
                    outT_v[k, pl.ds(c * 16, 16)] = g
            pltpu.sync_copy(outT_v, out_hbm.at[:, pl.ds(base, _B_PER_TEC)])

    return sc_gather


_BI = 256
_BJ = 512


def _dist_body(norm_ref, sta_ref, post_ref, out_ref):
    sa = sta_ref[...]
    aabs = jnp.abs(sa)
    asig = sa >> 31
    pt = post_ref[...]
    pabs = jnp.abs(pt)
    psig = pt >> 31
    acc = jnp.zeros((_BI, _BJ), jnp.int32)
    for k in range(TP):
        x = aabs[:, k : k + 1] ^ pabs[k : k + 1, :]
        q = lax.clz(x + 1) - TP
        m = asig[:, k : k + 1] ^ psig[k : k + 1, :]
        acc = acc + ((q ^ m) - m)
    out_ref[...] = acc.astype(jnp.float32) * (norm_ref[...] * (1.0 / 256.0))


def _tc_distance(norm, sta, rows_t, interpret=False):
    grid = (T1 // _BI, T2 // _BJ)
    return pl.pallas_call(
        _dist_body,
        grid=grid,
        in_specs=[
            pl.BlockSpec((_BI, _BJ), lambda i, j: (i, j)),
            pl.BlockSpec((_BI, TP), lambda i, j: (i, jnp.int32(0))),
            pl.BlockSpec((TP, _BJ), lambda i, j: (jnp.int32(0), j + T1 // _BJ)),
        ],
        out_specs=pl.BlockSpec((_BI, _BJ), lambda i, j: (i, j)),
        out_shape=jax.ShapeDtypeStruct((T1, T2), jnp.float32),
        interpret=interpret,
    )(norm, sta, rows_t)


def kernel(norm, sta_idx, pos_idx, locations):
    loc_u_t = locations.astype(jnp.uint32).T
    idx_all = jnp.concatenate([sta_idx, pos_idx]).astype(jnp.int32)
    rows_t = _make_sc_gather()(loc_u_t, idx_all)
    sta = rows_t[:, :T1].T
    return _tc_distance(norm, sta, rows_t)


# speedup vs baseline: 38.7055x; 1.0130x over previous
import functools

import jax
import jax.numpy as jnp
from jax import lax
from jax.experimental import pallas as pl
from jax.experimental.pallas import tpu as pltpu
from jax.experimental.pallas import tpu_sc as plsc

H = 16
TP = 16
T1 = 1024
T2 = 2048
B_ALL = T1 + T2

_NC = 2
_NS = 16
_B_PER_TEC = 128
_N_ACTIVE = B_ALL // _B_PER_TEC


def _make_sc_gather():
    mesh = plsc.VectorSubcoreMesh(core_axis_name="c", subcore_axis_name="s")

    @functools.partial(
        pl.kernel,
        mesh=mesh,
        out_type=jax.ShapeDtypeStruct((TP, B_ALL), jnp.int32),
        scratch_types=[
            pltpu.VMEM((_B_PER_TEC,), jnp.int32),
            pltpu.VMEM((2, 16, TP, 128), jnp.int32),
            pltpu.VMEM((TP, _B_PER_TEC), jnp.int32),
            pltpu.SemaphoreType.DMA,
        ],
        compiler_params=pltpu.CompilerParams(needs_layout_passes=False),
    )
    def sc_gather(table_hbm, idx_hbm, out_hbm, idx_v, bufs, outT_v, sem):
        wid = lax.axis_index("s") * _NC + lax.axis_index("c")

        @pl.when(wid < _N_ACTIVE)
        def _():
            base = wid * _B_PER_TEC
            pltpu.sync_copy(idx_hbm.at[pl.ds(base, _B_PER_TEC)], idx_v)
            lane = jnp.arange(16, dtype=jnp.int32)
            nchunk = _B_PER_TEC // 16

            def fire(c):
                chunk = idx_v[pl.ds(c * 16, 16)]
                tcol = (chunk >> 7) * 128
                for j in range(16):
                    bj = jnp.sum(jnp.where(lane == j, tcol, 0),
                                 dtype=jnp.int32)
                    bj = pl.multiple_of(bj, 128)
                    pltpu.make_async_copy(
                        table_hbm.at[pl.ds(jnp.int32(0), TP), pl.ds(bj, 128)],
                        bufs.at[jnp.int32(c % 2), jnp.int32(j)], sem,
                    ).start()

            def drain_extract(c):
                for j in range(16):
                    pltpu.make_async_copy(
                        table_hbm.at[pl.ds(jnp.int32(0), TP),
                                     pl.ds(jnp.int32(0), 128)],
                        bufs.at[jnp.int32(c % 2), jnp.int32(j)], sem,
                    ).wait()
                loff = idx_v[pl.ds(c * 16, 16)] & 127
                half = jnp.full_like(lane, c % 2)
                for k in range(TP):
                    g = plsc.load_gather(
                        bufs, [half, lane, jnp.full_like(lane, k), loff]
                    )
                    outT_v[k, pl.ds(c * 16, 16)] = g

            fire(0)
            for c in range(nchunk):
                if c + 1 < nchunk:
                    fire(c + 1)
                drain_extract(c)
            pltpu.sync_copy(outT_v, out_hbm.at[:, pl.ds(base, _B_PER_TEC)])

    return sc_gather


_BI = 256
_BJ = 512


def _dist_body(norm_ref, sta_ref, post_ref, out_ref):
    sa = sta_ref[...].T
    aabs = jnp.abs(sa)
    asig = sa >> 31
    pt = post_ref[...]
    pabs = jnp.abs(pt)
    psig = pt >> 31
    acc = jnp.zeros((_BI, _BJ), jnp.int32)
    for k in range(TP):
        x = aabs[:, k : k + 1] ^ pabs[k : k + 1, :]
        q = lax.clz(x + 1) - TP
        m = asig[:, k : k + 1] ^ psig[k : k + 1, :]
        acc = acc + ((q ^ m) - m)
    out_ref[...] = acc.astype(jnp.float32) * (norm_ref[...] * (1.0 / 256.0))


def _tc_distance(norm, sta_t, rows_t, interpret=False):
    grid = (T1 // _BI, T2 // _BJ)
    return pl.pallas_call(
        _dist_body,
        grid=grid,
        in_specs=[
            pl.BlockSpec((_BI, _BJ), lambda i, j: (i, j)),
            pl.BlockSpec((TP, _BI), lambda i, j: (jnp.int32(0), i)),
            pl.BlockSpec((TP, _BJ), lambda i, j: (jnp.int32(0), j + T1 // _BJ)),
        ],
        out_specs=pl.BlockSpec((_BI, _BJ), lambda i, j: (i, j)),
        out_shape=jax.ShapeDtypeStruct((T1, T2), jnp.float32),
        interpret=interpret,
    )(norm, sta_t, rows_t)


def kernel(norm, sta_idx, pos_idx, locations):
    loc_u_t = locations.astype(jnp.uint32).T
    idx_all = jnp.concatenate([sta_idx, pos_idx]).astype(jnp.int32)
    rows_t = _make_sc_gather()(loc_u_t, idx_all)
    return _tc_distance(norm, rows_t, rows_t)
